# baseline (device time: 137887 ns/iter reference)
import functools

import jax
import jax.numpy as jnp
from jax import lax
from jax.experimental import pallas as pl
from jax.experimental.pallas import tpu as pltpu

N_DEV = 32
B, SQ, D = 4, 256, 1024
HQ_LOC, HKV_LOC, DH = 8, 2, 128
SKV = 1024
SCALE = 0.08838834764831843
ROWS = B * SQ
HALF = ROWS // 2
QCH = HALF // 4
ZCH = QCH // 4

_MESH = pl.DeviceIdType.MESH

_RSX = 0
_RSY = 4
_RSZ = 7
_AGZ = 10
_AGY = 13
_AGX = 16
_NSEM = 20


def _logi(xx, yy, zz):
    return zz * 8 + yy * 2 + jnp.where(yy % 2 == 0, xx, 1 - xx)


def _body(x_ref, wq_ref, wo_ref, k_ref, v_ref, out_bf,
          acc_ref, q_ref, rbx, rby, rbz, send_sems, recv_sems):
    lid = lax.axis_index("i")
    z = lid // 8
    r = lid % 8
    y = r // 2
    xb = jnp.where(y % 2 == 0, r % 2, 1 - (r % 2))

    px = _logi(1 - xb, y, z)
    y_r = _logi(xb, (y + 1) % 4, z)
    y_l = _logi(xb, (y - 1) % 4, z)
    z_r = (lid + 8) % N_DEV
    z_l = (lid - 8) % N_DEV
    partners = (px, y_l, y_r, z_l, z_r)

    def qbase(c, xx):
        return (xx + 2 * (c // 2)) * SQ + (c % 2) * QCH

    barrier = pltpu.get_barrier_semaphore()
    for p in partners:
        pl.semaphore_signal(barrier, inc=1, device_id=(p,),
                            device_id_type=_MESH)
    pl.semaphore_wait(barrier, len(partners))

    def rdma(sem_idx, src, dst, target):
        return pltpu.make_async_remote_copy(
            src_ref=src, dst_ref=dst,
            send_sem=send_sems.at[sem_idx], recv_sem=recv_sems.at[sem_idx],
            device_id=(target,), device_id_type=_MESH,
        )

    xrd = []
    for t in range(4):
        c = (y - t) % 4
        xrd.append(rdma(_RSX + t,
                        acc_ref.at[pl.ds(qbase(c, 1 - xb), QCH), :],
                        rbx.at[pl.ds(c * QCH, QCH), :], px))

    x_b = x_ref[...].astype(jnp.bfloat16)
    wq_b = wq_ref[...].astype(jnp.bfloat16)
    q = jnp.dot(x_b, wq_b, preferred_element_type=jnp.float32)
    q_ref[...] = (q * SCALE).astype(jnp.bfloat16)
    wo_b = wo_ref[...].astype(jnp.bfloat16)

    for b in range(B):
        rows = slice(b * SQ, (b + 1) * SQ)
        acc = None
        for g in range(HKV_LOC):
            k_bg = k_ref[b, :, g, :].astype(jnp.bfloat16)
            v_bg = v_ref[b, :, g, :].astype(jnp.bfloat16)
            for hh in range(HQ_LOC // HKV_LOC):
                h = g * (HQ_LOC // HKV_LOC) + hh
                q_bh = q_ref[rows, h * DH:(h + 1) * DH]
                s = lax.dot_general(q_bh, k_bg, (((1,), (1,)), ((), ())),
                                    preferred_element_type=jnp.float32)
                m = jnp.max(s, axis=1, keepdims=True)
                p = jnp.exp(s - m)
                l = jnp.sum(p, axis=1, keepdims=True)
                o = jnp.dot(p.astype(jnp.bfloat16), v_bg,
                            preferred_element_type=jnp.float32)
                o = (o / l).astype(jnp.bfloat16)
                contrib = jnp.dot(o, wo_b[h * DH:(h + 1) * DH, :],
                                  preferred_element_type=jnp.float32)
                acc = contrib if acc is None else acc + contrib
        acc_ref[rows, :] = acc

        for t in range(4):
            c = (y - t) % 4
            src_batch = (1 - xb) + 2 * (c // 2)

            @pl.when(src_batch == b)
            def _(t=t):
                xrd[t].start()

    for t in range(4):
        xrd[t].wait()
        c = (y - t) % 4
        mb = qbase(c, xb)
        acc_ref[pl.ds(mb, QCH), :] = (
            acc_ref[pl.ds(mb, QCH), :] + rbx[pl.ds(c * QCH, QCH), :])
        if t < 3:
            qr = (y - t - 1) % 4
            rd = rdma(_RSY + t, acc_ref.at[pl.ds(mb, QCH), :], rby.at[t], y_r)
            rd.start()
            rd.wait()
            acc_ref[pl.ds(qbase(qr, xb), QCH), :] = (
                acc_ref[pl.ds(qbase(qr, xb), QCH), :] + rby[t, :, :])
    base1 = qbase((y + 1) % 4, xb)

    for t in range(3):
        cs = (z - t) % 4
        cr = (z - t - 1) % 4
        rd = rdma(_RSZ + t, acc_ref.at[pl.ds(base1 + cs * ZCH, ZCH), :],
                  rbz.at[t], z_r)
        rd.start()
        rd.wait()
        acc_ref[pl.ds(base1 + cr * ZCH, ZCH), :] = (
            acc_ref[pl.ds(base1 + cr * ZCH, ZCH), :] + rbz[t, :, :])

    bf = base1 + ((z + 1) % 4) * ZCH
    out_bf[pl.ds(bf, ZCH), :] = acc_ref[pl.ds(bf, ZCH), :].astype(jnp.bfloat16)

    for t in range(3):
        cc = (z + 1 - t) % 4
        sl = out_bf.at[pl.ds(base1 + cc * ZCH, ZCH), :]
        rd = rdma(_AGZ + t, sl, sl, z_r)
        rd.start()
        rd.wait()

    agx = []
    for t in range(3):
        blk = (y + 1 - t) % 4
        sl = out_bf.at[pl.ds(qbase(blk, xb), QCH), :]
        rdx = rdma(_AGX + t, sl, sl, px)
        rdx.start()
        agx.append(rdx)
        rdy = rdma(_AGY + t, sl, sl, y_r)
        rdy.start()
        rdy.wait()
    blk3 = (y - 2) % 4
    sl = out_bf.at[pl.ds(qbase(blk3, xb), QCH), :]
    rdx = rdma(_AGX + 3, sl, sl, px)
    rdx.start()
    agx.append(rdx)
    for rd in agx:
        rd.wait()

    @functools.partial(pl.run_scoped, sb=pltpu.SemaphoreType.REGULAR)
    def _(sb):
        for p in partners:
            pl.semaphore_signal(sb, inc=1, device_id=(p,),
                                device_id_type=_MESH)
        pl.semaphore_wait(sb, len(partners))


def kernel(x, Wq, Wo, K_ext, V_ext):
    i = lax.axis_index("i")

    K_sl = lax.dynamic_slice_in_dim(K_ext, 2 * i, HKV_LOC, axis=2)
    V_sl = lax.dynamic_slice_in_dim(V_ext, 2 * i, HKV_LOC, axis=2)
    x2 = x.reshape(ROWS, D)

    out = pl.pallas_call(
        _body,
        out_shape=jax.ShapeDtypeStruct((ROWS, D), jnp.bfloat16),
        in_specs=[pl.BlockSpec(memory_space=pltpu.VMEM)] * 5,
        out_specs=pl.BlockSpec(memory_space=pltpu.VMEM),
        scratch_shapes=[
            pltpu.VMEM((ROWS, D), jnp.float32),
            pltpu.VMEM((ROWS, HQ_LOC * DH), jnp.bfloat16),
            pltpu.VMEM((HALF, D), jnp.float32),
            pltpu.VMEM((3, QCH, D), jnp.float32),
            pltpu.VMEM((3, ZCH, D), jnp.float32),
            pltpu.SemaphoreType.DMA((_NSEM,)),
            pltpu.SemaphoreType.DMA((_NSEM,)),
        ],
        compiler_params=pltpu.CompilerParams(collective_id=0),
    )(x2, Wq, Wo, K_sl, V_sl)
    return out.reshape(B, SQ, D)


# device time: 117879 ns/iter; 1.1697x vs baseline; 1.1697x over previous
import functools

import jax
import jax.numpy as jnp
from jax import lax
from jax.experimental import pallas as pl
from jax.experimental.pallas import tpu as pltpu

N_DEV = 32
B, SQ, D = 4, 256, 1024
HQ_LOC, HKV_LOC, DH = 8, 2, 128
SKV = 1024
SCALE = 0.08838834764831843
ROWS = B * SQ
HALF = ROWS // 2
QCH = HALF // 4
ZCH = QCH // 4

_MESH = pl.DeviceIdType.MESH

_RSX = 0
_RSY = 4
_RSZ = 7
_AGZ = 10
_AGY = 13
_AGX = 16
_NSEM = 20


def _logi(xx, yy, zz):
    return zz * 8 + yy * 2 + jnp.where(yy % 2 == 0, xx, 1 - xx)


def _body(x_ref, wq_ref, wo_ref, kt_ref, vt_ref, out_bf,
          acc_ref, q_ref, sbx, rbx, sby, rby, sbz, rbz,
          send_sems, recv_sems):
    lid = lax.axis_index("i")
    z = lid // 8
    r = lid % 8
    y = r // 2
    xb = jnp.where(y % 2 == 0, r % 2, 1 - (r % 2))

    px = _logi(1 - xb, y, z)
    y_r = _logi(xb, (y + 1) % 4, z)
    y_l = _logi(xb, (y - 1) % 4, z)
    z_r = (lid + 8) % N_DEV
    z_l = (lid - 8) % N_DEV
    partners = (px, y_l, y_r, z_l, z_r)

    barrier = pltpu.get_barrier_semaphore()
    for p in partners:
        pl.semaphore_signal(barrier, inc=1, device_id=(p,),
                            device_id_type=_MESH)
    pl.semaphore_wait(barrier, len(partners))

    q = jnp.dot(x_ref[...], wq_ref[...], preferred_element_type=jnp.float32)
    q_ref[...] = (q * SCALE).astype(jnp.bfloat16)

    for b in range(B):
        rows = slice(b * SQ, (b + 1) * SQ)
        acc = None
        for g in range(HKV_LOC):
            k_bg = kt_ref[b, g, :, :]
            v_bg = vt_ref[b, g, :, :]
            for hh in range(HQ_LOC // HKV_LOC):
                h = g * (HQ_LOC // HKV_LOC) + hh
                q_bh = q_ref[rows, h * DH:(h + 1) * DH]
                s = jnp.dot(q_bh, k_bg, preferred_element_type=jnp.float32)
                m = jnp.max(s, axis=1, keepdims=True)
                p = jnp.exp(s - m)
                l = jnp.sum(p, axis=1, keepdims=True)
                o = jnp.dot(p.astype(jnp.bfloat16), v_bg,
                            preferred_element_type=jnp.float32)
                o = (o / l).astype(jnp.bfloat16)
                contrib = jnp.dot(o, wo_ref[h * DH:(h + 1) * DH, :],
                                  preferred_element_type=jnp.float32)
                acc = contrib if acc is None else acc + contrib
        acc_ref[rows, :] = acc

    def rdma(sem_idx, src, dst, target):
        return pltpu.make_async_remote_copy(
            src_ref=src, dst_ref=dst,
            send_sem=send_sems.at[sem_idx], recv_sem=recv_sems.at[sem_idx],
            device_id=(target,), device_id_type=_MESH,
        )

    hm = xb * HALF
    hp = (1 - xb) * HALF

    sbx[...] = acc_ref[pl.ds(hp, HALF), :].astype(jnp.bfloat16)
    xrd = []
    for t in range(4):
        c = (y - t) % 4
        rd = rdma(_RSX + t, sbx.at[pl.ds(c * QCH, QCH), :],
                  rbx.at[pl.ds(c * QCH, QCH), :], px)
        rd.start()
        xrd.append(rd)

    for t in range(4):
        xrd[t].wait()
        c = (y - t) % 4
        acc_ref[pl.ds(hm + c * QCH, QCH), :] = (
            acc_ref[pl.ds(hm + c * QCH, QCH), :]
            + rbx[pl.ds(c * QCH, QCH), :].astype(jnp.float32))
        if t < 3:
            qr = (y - t - 1) % 4
            sby[t, :, :] = acc_ref[pl.ds(hm + c * QCH, QCH), :].astype(
                jnp.bfloat16)
            rd = rdma(_RSY + t, sby.at[t], rby.at[t], y_r)
            rd.start()
            rd.wait()
            acc_ref[pl.ds(hm + qr * QCH, QCH), :] = (
                acc_ref[pl.ds(hm + qr * QCH, QCH), :]
                + rby[t, :, :].astype(jnp.float32))
    base1 = hm + ((y + 1) % 4) * QCH

    for t in range(3):
        cs = (z - t) % 4
        cr = (z - t - 1) % 4
        sbz[t, :, :] = acc_ref[pl.ds(base1 + cs * ZCH, ZCH), :].astype(
            jnp.bfloat16)
        rd = rdma(_RSZ + t, sbz.at[t], rbz.at[t], z_r)
        rd.start()
        rd.wait()
        acc_ref[pl.ds(base1 + cr * ZCH, ZCH), :] = (
            acc_ref[pl.ds(base1 + cr * ZCH, ZCH), :]
            + rbz[t, :, :].astype(jnp.float32))

    bf = base1 + ((z + 1) % 4) * ZCH
    out_bf[pl.ds(bf, ZCH), :] = acc_ref[pl.ds(bf, ZCH), :].astype(jnp.bfloat16)

    for t in range(3):
        cc = (z + 1 - t) % 4
        sl = out_bf.at[pl.ds(base1 + cc * ZCH, ZCH), :]
        rd = rdma(_AGZ + t, sl, sl, z_r)
        rd.start()
        rd.wait()

    agx = []
    for t in range(3):
        blk = (y + 1 - t) % 4
        sl = out_bf.at[pl.ds(hm + blk * QCH, QCH), :]
        rdx = rdma(_AGX + t, sl, sl, px)
        rdx.start()
        agx.append(rdx)
        rdy = rdma(_AGY + t, sl, sl, y_r)
        rdy.start()
        rdy.wait()
    blk3 = (y - 2) % 4
    sl = out_bf.at[pl.ds(hm + blk3 * QCH, QCH), :]
    rdx = rdma(_AGX + 3, sl, sl, px)
    rdx.start()
    agx.append(rdx)
    for rd in agx:
        rd.wait()

    @functools.partial(pl.run_scoped, sb=pltpu.SemaphoreType.REGULAR)
    def _(sb):
        for p in partners:
            pl.semaphore_signal(sb, inc=1, device_id=(p,),
                                device_id_type=_MESH)
        pl.semaphore_wait(sb, len(partners))


def kernel(x, Wq, Wo, K_ext, V_ext):
    i = lax.axis_index("i")

    K_sl = lax.dynamic_slice_in_dim(K_ext, 2 * i, HKV_LOC, axis=2)
    V_sl = lax.dynamic_slice_in_dim(V_ext, 2 * i, HKV_LOC, axis=2)
    kt = jnp.transpose(K_sl, (0, 2, 3, 1)).astype(jnp.bfloat16)
    vt = jnp.transpose(V_sl, (0, 2, 1, 3)).astype(jnp.bfloat16)
    x2 = x.reshape(ROWS, D).astype(jnp.bfloat16)
    wq = Wq.astype(jnp.bfloat16)
    wo = Wo.astype(jnp.bfloat16)

    out = pl.pallas_call(
        _body,
        out_shape=jax.ShapeDtypeStruct((ROWS, D), jnp.bfloat16),
        in_specs=[pl.BlockSpec(memory_space=pltpu.VMEM)] * 5,
        out_specs=pl.BlockSpec(memory_space=pltpu.VMEM),
        scratch_shapes=[
            pltpu.VMEM((ROWS, D), jnp.float32),
            pltpu.VMEM((ROWS, HQ_LOC * DH), jnp.bfloat16),
            pltpu.VMEM((HALF, D), jnp.bfloat16),
            pltpu.VMEM((HALF, D), jnp.bfloat16),
            pltpu.VMEM((3, QCH, D), jnp.bfloat16),
            pltpu.VMEM((3, QCH, D), jnp.bfloat16),
            pltpu.VMEM((3, ZCH, D), jnp.bfloat16),
            pltpu.VMEM((3, ZCH, D), jnp.bfloat16),
            pltpu.SemaphoreType.DMA((_NSEM,)),
            pltpu.SemaphoreType.DMA((_NSEM,)),
        ],
        compiler_params=pltpu.CompilerParams(collective_id=0),
    )(x2, wq, wo, kt, vt)
    return out.reshape(B, SQ, D)


# device time: 117831 ns/iter; 1.1702x vs baseline; 1.0004x over previous
import functools

import jax
import jax.numpy as jnp
from jax import lax
from jax.experimental import pallas as pl
from jax.experimental.pallas import tpu as pltpu

N_DEV = 32
B, SQ, D = 4, 256, 1024
HQ_LOC, HKV_LOC, DH = 8, 2, 128
SKV = 1024
SCALE = 0.08838834764831843
ROWS = B * SQ
HALF = ROWS // 2
QCH = HALF // 4
ZCH = QCH // 4

_MESH = pl.DeviceIdType.MESH

_RSX = 0
_RSY = 4
_RSZ = 7
_AGZ = 10
_AGY = 13
_AGX = 16
_NSEM = 20


def _logi(xx, yy, zz):
    return zz * 8 + yy * 2 + jnp.where(yy % 2 == 0, xx, 1 - xx)


def _body(x_ref, wq_ref, wo_ref, kt_ref, vt_ref, out_bf,
          acc_ref, q_ref, sbx, rbx, sby, rby, sbz, rbz,
          send_sems, recv_sems):
    lid = lax.axis_index("i")
    z = lid // 8
    r = lid % 8
    y = r // 2
    xb = jnp.where(y % 2 == 0, r % 2, 1 - (r % 2))

    px = _logi(1 - xb, y, z)
    y_r = _logi(xb, (y + 1) % 4, z)
    y_l = _logi(xb, (y - 1) % 4, z)
    z_r = (lid + 8) % N_DEV
    z_l = (lid - 8) % N_DEV
    partners = (px, y_l, y_r, z_l, z_r)

    barrier = pltpu.get_barrier_semaphore()
    for p in partners:
        pl.semaphore_signal(barrier, inc=1, device_id=(p,),
                            device_id_type=_MESH)
    pl.semaphore_wait(barrier, len(partners))

    q = jnp.dot(x_ref[...], wq_ref[...], preferred_element_type=jnp.float32)
    q_ref[...] = (q * SCALE).astype(jnp.bfloat16)

    for b in range(B):
        rows = slice(b * SQ, (b + 1) * SQ)
        acc = None
        for g in range(HKV_LOC):
            k_bg = kt_ref[b, g, :, :]
            v_bg = vt_ref[b, g, :, :]
            for hh in range(HQ_LOC // HKV_LOC):
                h = g * (HQ_LOC // HKV_LOC) + hh
                q_bh = q_ref[rows, h * DH:(h + 1) * DH]
                s = jnp.dot(q_bh, k_bg, preferred_element_type=jnp.float32)
                m = jnp.max(s, axis=1, keepdims=True)
                p = jnp.exp(s - m)
                l = jnp.sum(p, axis=1, keepdims=True)
                o = jnp.dot(p.astype(jnp.bfloat16), v_bg,
                            preferred_element_type=jnp.float32)
                o = (o / l).astype(jnp.bfloat16)
                contrib = jnp.dot(o, wo_ref[h * DH:(h + 1) * DH, :],
                                  preferred_element_type=jnp.float32)
                acc = contrib if acc is None else acc + contrib
        acc_ref[rows, :] = acc

    def rdma(sem_idx, src, dst, target):
        return pltpu.make_async_remote_copy(
            src_ref=src, dst_ref=dst,
            send_sem=send_sems.at[sem_idx], recv_sem=recv_sems.at[sem_idx],
            device_id=(target,), device_id_type=_MESH,
        )

    hm = xb * HALF
    hp = (1 - xb) * HALF

    sbx[...] = acc_ref[pl.ds(hp, HALF), :].astype(jnp.bfloat16)
    xrd = []
    for t in range(4):
        c = (y - t) % 4
        rd = rdma(_RSX + t, sbx.at[pl.ds(c * QCH, QCH), :],
                  rbx.at[pl.ds(c * QCH, QCH), :], px)
        rd.start()
        xrd.append(rd)

    for t in range(4):
        xrd[t].wait()
        c = (y - t) % 4
        acc_ref[pl.ds(hm + c * QCH, QCH), :] = (
            acc_ref[pl.ds(hm + c * QCH, QCH), :]
            + rbx[pl.ds(c * QCH, QCH), :].astype(jnp.float32))
        if t < 3:
            qr = (y - t - 1) % 4
            sby[t, :, :] = acc_ref[pl.ds(hm + c * QCH, QCH), :].astype(
                jnp.bfloat16)
            rd = rdma(_RSY + t, sby.at[t], rby.at[t], y_r)
            rd.start()
            rd.wait()
            acc_ref[pl.ds(hm + qr * QCH, QCH), :] = (
                acc_ref[pl.ds(hm + qr * QCH, QCH), :]
                + rby[t, :, :].astype(jnp.float32))
    base1 = hm + ((y + 1) % 4) * QCH

    for t in range(3):
        cs = (z - t) % 4
        cr = (z - t - 1) % 4
        sbz[t, :, :] = acc_ref[pl.ds(base1 + cs * ZCH, ZCH), :].astype(
            jnp.bfloat16)
        rd = rdma(_RSZ + t, sbz.at[t], rbz.at[t], z_r)
        rd.start()
        rd.wait()
        acc_ref[pl.ds(base1 + cr * ZCH, ZCH), :] = (
            acc_ref[pl.ds(base1 + cr * ZCH, ZCH), :]
            + rbz[t, :, :].astype(jnp.float32))

    bf = base1 + ((z + 1) % 4) * ZCH
    out_bf[pl.ds(bf, ZCH), :] = acc_ref[pl.ds(bf, ZCH), :].astype(jnp.bfloat16)

    for t in range(3):
        cc = (z + 1 - t) % 4
        sl = out_bf.at[pl.ds(base1 + cc * ZCH, ZCH), :]
        rd = rdma(_AGZ + t, sl, sl, z_r)
        rd.start()
        rd.wait()

    agx = []
    for t in range(3):
        blk = (y + 1 - t) % 4
        sl = out_bf.at[pl.ds(hm + blk * QCH, QCH), :]
        rdx = rdma(_AGX + t, sl, sl, px)
        rdx.start()
        agx.append(rdx)
        rdy = rdma(_AGY + t, sl, sl, y_r)
        rdy.start()
        rdy.wait()
    blk3 = (y - 2) % 4
    sl = out_bf.at[pl.ds(hm + blk3 * QCH, QCH), :]
    rdx = rdma(_AGX + 3, sl, sl, px)
    rdx.start()
    agx.append(rdx)
    for rd in agx:
        rd.wait()

    @functools.partial(pl.run_scoped, sb=pltpu.SemaphoreType.REGULAR)
    def _(sb):
        for p in partners:
            pl.semaphore_signal(sb, inc=1, device_id=(p,),
                                device_id_type=_MESH)
        pl.semaphore_wait(sb, len(partners))


def kernel(x, Wq, Wo, K_ext, V_ext):
    i = lax.axis_index("i")

    K_sl = lax.dynamic_slice_in_dim(K_ext, 2 * i, HKV_LOC, axis=2)
    V_sl = lax.dynamic_slice_in_dim(V_ext, 2 * i, HKV_LOC, axis=2)
    K_sl, V_sl = lax.optimization_barrier(
        (K_sl.astype(jnp.bfloat16), V_sl.astype(jnp.bfloat16)))
    kt = jnp.transpose(K_sl, (0, 2, 3, 1))
    vt = jnp.transpose(V_sl, (0, 2, 1, 3))
    x2 = x.reshape(ROWS, D).astype(jnp.bfloat16)
    wq = Wq.astype(jnp.bfloat16)
    wo = Wo.astype(jnp.bfloat16)

    out = pl.pallas_call(
        _body,
        out_shape=jax.ShapeDtypeStruct((ROWS, D), jnp.bfloat16),
        in_specs=[pl.BlockSpec(memory_space=pltpu.VMEM)] * 5,
        out_specs=pl.BlockSpec(memory_space=pltpu.VMEM),
        scratch_shapes=[
            pltpu.VMEM((ROWS, D), jnp.float32),
            pltpu.VMEM((ROWS, HQ_LOC * DH), jnp.bfloat16),
            pltpu.VMEM((HALF, D), jnp.bfloat16),
            pltpu.VMEM((HALF, D), jnp.bfloat16),
            pltpu.VMEM((3, QCH, D), jnp.bfloat16),
            pltpu.VMEM((3, QCH, D), jnp.bfloat16),
            pltpu.VMEM((3, ZCH, D), jnp.bfloat16),
            pltpu.VMEM((3, ZCH, D), jnp.bfloat16),
            pltpu.SemaphoreType.DMA((_NSEM,)),
            pltpu.SemaphoreType.DMA((_NSEM,)),
        ],
        compiler_params=pltpu.CompilerParams(collective_id=0),
    )(x2, wq, wo, kt, vt)
    return out.reshape(B, SQ, D)


# device time: 94725 ns/iter; 1.4557x vs baseline; 1.2439x over previous
import functools

import jax
import jax.numpy as jnp
from jax import lax
from jax.experimental import pallas as pl
from jax.experimental.pallas import tpu as pltpu

N_DEV = 32
B, SQ, D = 4, 256, 1024
HQ_LOC, HKV_LOC, DH = 8, 2, 128
SKV = 1024
SCALE = 0.08838834764831843
ROWS = B * SQ
HALF = ROWS // 2
QCH = HALF // 4
ZCH = QCH // 4

_MESH = pl.DeviceIdType.MESH

_RSX = 0
_RSY = 4
_RSZ = 7
_AGZ = 10
_AGY = 13
_AGX = 16
_NSEM = 20


def _logi(xx, yy, zz):
    return zz * 8 + yy * 2 + jnp.where(yy % 2 == 0, xx, 1 - xx)


def _body(x_ref, wq_ref, wo_ref, k_hbm, v_hbm, out_bf,
          acc_ref, q_ref, kbuf, vbuf, sbx, rbx, sby, rby, sbz, rbz,
          copy_sems, send_sems, recv_sems):
    lid = lax.axis_index("i")
    z = lid // 8
    r = lid % 8
    y = r // 2
    xb = jnp.where(y % 2 == 0, r % 2, 1 - (r % 2))

    px = _logi(1 - xb, y, z)
    y_r = _logi(xb, (y + 1) % 4, z)
    y_l = _logi(xb, (y - 1) % 4, z)
    z_r = (lid + 8) % N_DEV
    z_l = (lid - 8) % N_DEV
    partners = (px, y_l, y_r, z_l, z_r)

    kv_copies = []
    for b in range(B):
        for g in range(HKV_LOC):
            hidx = 2 * lid + g
            ck = pltpu.make_async_copy(
                k_hbm.at[b, :, hidx, :], kbuf.at[b, g],
                copy_sems.at[b * HKV_LOC + g])
            cv = pltpu.make_async_copy(
                v_hbm.at[b, :, hidx, :], vbuf.at[b, g],
                copy_sems.at[B * HKV_LOC + b * HKV_LOC + g])
            ck.start()
            cv.start()
            kv_copies.append((ck, cv))

    barrier = pltpu.get_barrier_semaphore()
    for p in partners:
        pl.semaphore_signal(barrier, inc=1, device_id=(p,),
                            device_id_type=_MESH)
    pl.semaphore_wait(barrier, len(partners))

    q = jnp.dot(x_ref[...], wq_ref[...], preferred_element_type=jnp.float32)
    q_ref[...] = (q * SCALE).astype(jnp.bfloat16)

    for b in range(B):
        rows = slice(b * SQ, (b + 1) * SQ)
        acc = None
        for g in range(HKV_LOC):
            ck, cv = kv_copies[b * HKV_LOC + g]
            ck.wait()
            cv.wait()
            k_bg = kbuf[b, g, :, :].astype(jnp.bfloat16)
            v_bg = vbuf[b, g, :, :].astype(jnp.bfloat16)
            for hh in range(HQ_LOC // HKV_LOC):
                h = g * (HQ_LOC // HKV_LOC) + hh
                q_bh = q_ref[rows, h * DH:(h + 1) * DH]
                s = lax.dot_general(q_bh, k_bg, (((1,), (1,)), ((), ())),
                                    preferred_element_type=jnp.float32)
                m = jnp.max(s, axis=1, keepdims=True)
                p = jnp.exp(s - m)
                l = jnp.sum(p, axis=1, keepdims=True)
                o = jnp.dot(p.astype(jnp.bfloat16), v_bg,
                            preferred_element_type=jnp.float32)
                o = (o / l).astype(jnp.bfloat16)
                contrib = jnp.dot(o, wo_ref[h * DH:(h + 1) * DH, :],
                                  preferred_element_type=jnp.float32)
                acc = contrib if acc is None else acc + contrib
        acc_ref[rows, :] = acc

    def rdma(sem_idx, src, dst, target):
        return pltpu.make_async_remote_copy(
            src_ref=src, dst_ref=dst,
            send_sem=send_sems.at[sem_idx], recv_sem=recv_sems.at[sem_idx],
            device_id=(target,), device_id_type=_MESH,
        )

    hm = xb * HALF
    hp = (1 - xb) * HALF

    sbx[...] = acc_ref[pl.ds(hp, HALF), :].astype(jnp.bfloat16)
    xrd = []
    for t in range(4):
        c = (y - t) % 4
        rd = rdma(_RSX + t, sbx.at[pl.ds(c * QCH, QCH), :],
                  rbx.at[pl.ds(c * QCH, QCH), :], px)
        rd.start()
        xrd.append(rd)

    for t in range(4):
        xrd[t].wait()
        c = (y - t) % 4
        acc_ref[pl.ds(hm + c * QCH, QCH), :] = (
            acc_ref[pl.ds(hm + c * QCH, QCH), :]
            + rbx[pl.ds(c * QCH, QCH), :].astype(jnp.float32))
        if t < 3:
            qr = (y - t - 1) % 4
            sby[t, :, :] = acc_ref[pl.ds(hm + c * QCH, QCH), :].astype(
                jnp.bfloat16)
            rd = rdma(_RSY + t, sby.at[t], rby.at[t], y_r)
            rd.start()
            rd.wait()
            acc_ref[pl.ds(hm + qr * QCH, QCH), :] = (
                acc_ref[pl.ds(hm + qr * QCH, QCH), :]
                + rby[t, :, :].astype(jnp.float32))
    base1 = hm + ((y + 1) % 4) * QCH

    for t in range(3):
        cs = (z - t) % 4
        cr = (z - t - 1) % 4
        sbz[t, :, :] = acc_ref[pl.ds(base1 + cs * ZCH, ZCH), :].astype(
            jnp.bfloat16)
        rd = rdma(_RSZ + t, sbz.at[t], rbz.at[t], z_r)
        rd.start()
        rd.wait()
        acc_ref[pl.ds(base1 + cr * ZCH, ZCH), :] = (
            acc_ref[pl.ds(base1 + cr * ZCH, ZCH), :]
            + rbz[t, :, :].astype(jnp.float32))

    bf = base1 + ((z + 1) % 4) * ZCH
    out_bf[pl.ds(bf, ZCH), :] = acc_ref[pl.ds(bf, ZCH), :].astype(jnp.bfloat16)

    for t in range(3):
        cc = (z + 1 - t) % 4
        sl = out_bf.at[pl.ds(base1 + cc * ZCH, ZCH), :]
        rd = rdma(_AGZ + t, sl, sl, z_r)
        rd.start()
        rd.wait()

    agx = []
    for t in range(3):
        blk = (y + 1 - t) % 4
        sl = out_bf.at[pl.ds(hm + blk * QCH, QCH), :]
        rdx = rdma(_AGX + t, sl, sl, px)
        rdx.start()
        agx.append(rdx)
        rdy = rdma(_AGY + t, sl, sl, y_r)
        rdy.start()
        rdy.wait()
    blk3 = (y - 2) % 4
    sl = out_bf.at[pl.ds(hm + blk3 * QCH, QCH), :]
    rdx = rdma(_AGX + 3, sl, sl, px)
    rdx.start()
    agx.append(rdx)
    for rd in agx:
        rd.wait()

    @functools.partial(pl.run_scoped, sb=pltpu.SemaphoreType.REGULAR)
    def _(sb):
        for p in partners:
            pl.semaphore_signal(sb, inc=1, device_id=(p,),
                                device_id_type=_MESH)
        pl.semaphore_wait(sb, len(partners))


def kernel(x, Wq, Wo, K_ext, V_ext):
    i = lax.axis_index("i")

    x2 = x.reshape(ROWS, D).astype(jnp.bfloat16)
    wq = Wq.astype(jnp.bfloat16)
    wo = Wo.astype(jnp.bfloat16)

    out = pl.pallas_call(
        _body,
        out_shape=jax.ShapeDtypeStruct((ROWS, D), jnp.bfloat16),
        in_specs=[pl.BlockSpec(memory_space=pltpu.VMEM)] * 3
        + [pl.BlockSpec(memory_space=pl.ANY)] * 2,
        out_specs=pl.BlockSpec(memory_space=pltpu.VMEM),
        scratch_shapes=[
            pltpu.VMEM((ROWS, D), jnp.float32),
            pltpu.VMEM((ROWS, HQ_LOC * DH), jnp.bfloat16),
            pltpu.VMEM((B, HKV_LOC, SKV, DH), jnp.float32),
            pltpu.VMEM((B, HKV_LOC, SKV, DH), jnp.float32),
            pltpu.VMEM((HALF, D), jnp.bfloat16),
            pltpu.VMEM((HALF, D), jnp.bfloat16),
            pltpu.VMEM((3, QCH, D), jnp.bfloat16),
            pltpu.VMEM((3, QCH, D), jnp.bfloat16),
            pltpu.VMEM((3, ZCH, D), jnp.bfloat16),
            pltpu.VMEM((3, ZCH, D), jnp.bfloat16),
            pltpu.SemaphoreType.DMA((2 * B * HKV_LOC,)),
            pltpu.SemaphoreType.DMA((_NSEM,)),
            pltpu.SemaphoreType.DMA((_NSEM,)),
        ],
        compiler_params=pltpu.CompilerParams(collective_id=0),
    )(x2, wq, wo, K_ext, V_ext)
    return out.reshape(B, SQ, D)
